# baseline (speedup 1.0000x reference)
import functools

import jax
import jax.numpy as jnp
from jax import lax
from jax.experimental import pallas as pl
from jax.experimental.pallas import tpu as pltpu
from jax.experimental.pallas import tpu_sc as plsc

D_MODEL = 128
SCALE = float(D_MODEL) ** 0.5

_NC = 2
_NS = 16
_NW = _NC * _NS

_LANES = 16
_NBUF = 8
_K = 5


def _make_kernel(batch: int, seq: int, seq_pad: int):
    assert batch % _NW == 0
    nch = batch // _NW
    assert nch >= _NBUF

    mesh = plsc.VectorSubcoreMesh(core_axis_name="c", subcore_axis_name="s")

    @functools.partial(
        pl.kernel,
        out_type=jax.ShapeDtypeStruct((batch, seq_pad, D_MODEL), jnp.float32),
        mesh=mesh,
        scratch_types=(
            [pltpu.VMEM((nch, seq), jnp.int32)]
            + [pltpu.VMEM((seq_pad, D_MODEL), jnp.float32)] * _NBUF
            + [pltpu.SemaphoreType.DMA] * (2 * _NBUF)
        ),
    )
    def emb_kernel(x_hbm, table_hbm, out_hbm, idx_v, *bufs_and_sems):
        rows = bufs_and_sems[:_NBUF]
        gsem = bufs_and_sems[_NBUF:2 * _NBUF]
        ssem = bufs_and_sems[2 * _NBUF:]

        wid = lax.axis_index("s") * _NC + lax.axis_index("c")
        row0 = wid * nch
        pltpu.sync_copy(x_hbm.at[pl.ds(row0, nch)], idx_v)

        def start_gather(j, b):
            pltpu.async_copy(
                table_hbm.at[idx_v.at[j]], rows[b].at[pl.ds(0, seq)], gsem[b])

        def wait_gather(b):
            pltpu.make_async_copy(
                table_hbm.at[idx_v.at[0]], rows[b].at[pl.ds(0, seq)],
                gsem[b]).wait()

        def start_store(j, b):
            pltpu.async_copy(rows[b], out_hbm.at[row0 + j], ssem[b])

        def wait_store(b):
            pltpu.make_async_copy(rows[b], out_hbm.at[row0], ssem[b]).wait()

        for b in range(_K):
            start_gather(b, b)

        def _scale(b):
            def scale_row(r, c2):
                for c in range(D_MODEL // _LANES):
                    sl = pl.ds(c * _LANES, _LANES)
                    rows[b][r, sl] = rows[b][r, sl] * SCALE
                return c2
            lax.fori_loop(0, seq, scale_row, 0)

        def outer(o, carry):
            for b in range(_NBUF):
                j = o * _NBUF + b
                jn = j + _K
                bn = (b + _K) % _NBUF

                @pl.when(jn < nch)
                def _():
                    @pl.when(j >= _NBUF - _K)
                    def _():
                        wait_store(bn)
                    start_gather(jn, bn)

                wait_gather(b)
                _scale(b)
                start_store(j, b)
            return carry

        assert nch % _NBUF == 0
        lax.fori_loop(0, nch // _NBUF, outer, 0)

        for b in range(_NBUF):
            wait_store(b)

    return emb_kernel


def kernel(x, table):
    b, s = x.shape
    s_pad = (s + 7) // 8 * 8
    padded = _make_kernel(b, s, s_pad)(x.astype(jnp.int32), table)
    return lax.slice(padded, (0, 0, 0), (b, s, D_MODEL))
